# baseline (speedup 1.0000x reference)
import functools

import jax
import jax.numpy as jnp
from jax import lax
from jax.experimental import pallas as pl
from jax.experimental.pallas import tpu as pltpu
from jax.experimental.pallas import tpu_sc as plsc

D = 32
NS = 5
NC = 2
NSUB = 16
NW = NC * NSUB
CH = 160
GRP = 16


def _sc_ctx_gather(ctx_table, oid, xid, r_total):
    rw = r_total // NW
    nchunk = rw // CH

    mesh = plsc.VectorSubcoreMesh(
        core_axis_name="c", subcore_axis_name="s",
        num_cores=NC, num_subcores=NSUB)

    @functools.partial(
        pl.kernel,
        out_type=(jax.ShapeDtypeStruct((r_total, D), jnp.float32),
                  jax.ShapeDtypeStruct((r_total * NS, D), jnp.float32)),
        mesh=mesh,
        compiler_params=pltpu.CompilerParams(
            needs_layout_passes=False, use_tc_tiling_on_sc=False,
            disable_bounds_checks=True),
        scratch_types=[
            pltpu.VMEM((rw,), jnp.int32),
            pltpu.VMEM((rw * NS,), jnp.int32),
            pltpu.VMEM((2, CH, D), jnp.float32),
            pltpu.VMEM((2, CH * NS, D), jnp.float32),
            pltpu.SemaphoreType.DMA,
            pltpu.SemaphoreType.DMA,
        ],
    )
    def body(ctx_hbm, oid_hbm, xid_hbm, ro_hbm, rx_hbm,
             oidx_v, xidx_v, out_v, noise_v, sem, sem_o):
        wid = lax.axis_index("s") * NC + lax.axis_index("c")
        pltpu.sync_copy(oid_hbm.at[pl.ds(wid * rw, rw)], oidx_v)
        pltpu.sync_copy(xid_hbm.at[pl.ds(wid * rw * NS, rw * NS)], xidx_v)

        def fire(c, slot):
            pltpu.async_copy(
                ctx_hbm.at[oidx_v.at[pl.ds(c * CH, CH)]],
                out_v.at[slot], sem)
            pltpu.async_copy(
                ctx_hbm.at[xidx_v.at[pl.ds(c * CH * NS, CH * NS)]],
                noise_v.at[slot], sem)

        def wait_in(c, slot):
            pltpu.make_async_copy(
                ctx_hbm.at[oidx_v.at[pl.ds(c * CH, CH)]],
                out_v.at[slot], sem).wait()
            pltpu.make_async_copy(
                ctx_hbm.at[xidx_v.at[pl.ds(c * CH * NS, CH * NS)]],
                noise_v.at[slot], sem).wait()

        def out_slices(c, slot):
            b0 = wid * rw + c * CH
            x0 = (wid * rw + c * CH) * NS
            return ((out_v.at[slot], ro_hbm.at[pl.ds(b0, CH)]),
                    (noise_v.at[slot], rx_hbm.at[pl.ds(x0, CH * NS)]))

        def wait_out(c, slot):
            for s_ref, d_ref in out_slices(c, slot):
                pltpu.make_async_copy(s_ref, d_ref, sem_o).wait()

        fire(0, 0)

        def chunk_body(c, carry):
            slot = lax.rem(c, 2)

            @pl.when(c >= 2)
            def _():
                wait_out(c - 2, slot)

            wait_in(c, slot)

            @pl.when(c + 1 < nchunk)
            def _():
                fire(c + 1, 1 - slot)

            for s_ref, d_ref in out_slices(c, slot):
                pltpu.async_copy(s_ref, d_ref, sem_o)
            return carry

        lax.fori_loop(0, nchunk, chunk_body, 0)

        def drain(c, carry):
            wait_out(c, lax.rem(c, 2))
            return carry

        lax.fori_loop(jnp.maximum(nchunk - 2, 0), nchunk, drain, 0)

    return body(ctx_table, oid, xid)


def _sc_logits(node_table, rows_out, rows_noise, nid, r_total):
    rw = r_total // NW
    nchunk = rw // CH

    mesh = plsc.VectorSubcoreMesh(
        core_axis_name="c", subcore_axis_name="s",
        num_cores=NC, num_subcores=NSUB)

    @functools.partial(
        pl.kernel,
        out_type=jax.ShapeDtypeStruct((6, r_total), jnp.float32),
        mesh=mesh,
        compiler_params=pltpu.CompilerParams(
            needs_layout_passes=False, use_tc_tiling_on_sc=False,
            disable_bounds_checks=True),
        scratch_types=[
            pltpu.VMEM((rw,), jnp.int32),
            pltpu.VMEM((2, CH, D), jnp.float32),
            pltpu.VMEM((2, CH, D), jnp.float32),
            pltpu.VMEM((2, CH * NS, D), jnp.float32),
            pltpu.VMEM((6 * rw,), jnp.float32),
            pltpu.SemaphoreType.DMA,
        ],
    )
    def body(node_hbm, ro_hbm, rx_hbm, nid_hbm, t_hbm,
             nidx_v, node_v, out_v, noise_v, t_v, sem):
        wid = lax.axis_index("s") * NC + lax.axis_index("c")
        pltpu.sync_copy(nid_hbm.at[pl.ds(wid * rw, rw)], nidx_v)

        lane = lax.iota(jnp.int32, GRP)

        def fire(c, slot):
            b0 = wid * rw + c * CH
            pltpu.async_copy(
                node_hbm.at[nidx_v.at[pl.ds(c * CH, CH)]],
                node_v.at[slot], sem)
            pltpu.async_copy(ro_hbm.at[pl.ds(b0, CH)], out_v.at[slot], sem)
            pltpu.async_copy(rx_hbm.at[pl.ds(b0 * NS, CH * NS)],
                             noise_v.at[slot], sem)

        def wait_chunk(c, slot):
            b0 = wid * rw + c * CH
            pltpu.make_async_copy(
                node_hbm.at[nidx_v.at[pl.ds(c * CH, CH)]],
                node_v.at[slot], sem).wait()
            pltpu.make_async_copy(
                ro_hbm.at[pl.ds(b0, CH)], out_v.at[slot], sem).wait()
            pltpu.make_async_copy(
                rx_hbm.at[pl.ds(b0 * NS, CH * NS)], noise_v.at[slot],
                sem).wait()

        fire(0, 0)

        def chunk_body(c, carry):
            slot = lax.rem(c, 2)

            @pl.when(c + 1 < nchunk)
            def _():
                fire(c + 1, 1 - slot)

            wait_chunk(c, slot)
            svec = jnp.full((GRP,), 0, jnp.int32) + slot

            def group_body(g, gcarry):
                row16 = g * GRP + lane
                nrows = [row16 * NS + s for s in range(NS)]
                accs = [jnp.zeros((GRP,), jnp.float32) for _ in range(6)]
                for d in range(D):
                    dcol = jnp.full((GRP,), d, jnp.int32)
                    vi = plsc.load_gather(node_v, [svec, row16, dcol])
                    vo = plsc.load_gather(out_v, [svec, row16, dcol])
                    accs[0] = accs[0] + vi * vo
                    for s in range(NS):
                        vn = plsc.load_gather(
                            noise_v, [svec, nrows[s], dcol])
                        accs[1 + s] = accs[1 + s] + vi * vn
                base = c * CH + g * GRP
                for k in range(6):
                    t_v[pl.ds(k * rw + base, GRP)] = accs[k]
                return gcarry

            lax.fori_loop(0, CH // GRP, group_body, 0)
            return carry

        lax.fori_loop(0, nchunk, chunk_body, 0)
        for k in range(6):
            pltpu.sync_copy(t_v.at[pl.ds(k * rw, rw)],
                            t_hbm.at[k, pl.ds(wid * rw, rw)])

    return body(node_table, rows_out, rows_noise, nid)


def _tc_reduce(t, batch):
    def body(t_ref, o_ref):
        x = t_ref[...]
        pos = x[0:1, :]
        neg = x[1:6, :]

        def logsig(z):
            return jnp.minimum(z, 0.0) - jnp.log(1.0 + jnp.exp(-jnp.abs(z)))

        total = jnp.sum(logsig(pos)) + jnp.sum(logsig(-neg))
        o_ref[0, 0] = -total / batch

    out = pl.pallas_call(
        body,
        out_shape=jax.ShapeDtypeStruct((1, 1), jnp.float32),
        out_specs=pl.BlockSpec(memory_space=pltpu.SMEM),
    )(t)
    return out[0, 0]


def kernel(input_labels, out_labels, noise_idx, num_sampled, node_table,
           ctx_table):
    b, w = out_labels.shape
    r_total = b * w
    nid = jnp.tile(input_labels.astype(jnp.int32), w)
    oid = out_labels.reshape(-1).astype(jnp.int32)
    xid = noise_idx.astype(jnp.int32).reshape(-1)
    rows_out, rows_noise = _sc_ctx_gather(ctx_table, oid, xid, r_total)
    t = _sc_logits(node_table, rows_out, rows_noise, nid, r_total)
    return _tc_reduce(t, b)


# speedup vs baseline: 1.0224x; 1.0224x over previous
import functools

import jax
import jax.numpy as jnp
from jax import lax
from jax.experimental import pallas as pl
from jax.experimental.pallas import tpu as pltpu
from jax.experimental.pallas import tpu_sc as plsc

D = 32
NS = 5
NC = 2
NSUB = 16
NW = NC * NSUB
CH = 160
GRP = 16


def _sc_logits(node_table, ctx_table, nid, oid, xid, r_total):
    rw = r_total // NW
    nchunk = rw // CH

    mesh = plsc.VectorSubcoreMesh(
        core_axis_name="c", subcore_axis_name="s",
        num_cores=NC, num_subcores=NSUB)

    @functools.partial(
        pl.kernel,
        out_type=jax.ShapeDtypeStruct((6, r_total), jnp.float32),
        mesh=mesh,
        compiler_params=pltpu.CompilerParams(
            needs_layout_passes=False, use_tc_tiling_on_sc=False,
            disable_bounds_checks=True),
        scratch_types=[
            pltpu.VMEM((rw,), jnp.int32),
            pltpu.VMEM((rw,), jnp.int32),
            pltpu.VMEM((rw * NS,), jnp.int32),
            pltpu.VMEM((2, CH, D), jnp.float32),
            pltpu.VMEM((2, CH, D), jnp.float32),
            pltpu.VMEM((2, CH * NS, D), jnp.float32),
            pltpu.VMEM((6 * rw,), jnp.float32),
            pltpu.SemaphoreType.DMA,
        ],
    )
    def body(node_hbm, ctx_hbm, nid_hbm, oid_hbm, xid_hbm, t_hbm,
             nidx_v, oidx_v, xidx_v, node_v, out_v, noise_v, t_v, sem):
        wid = lax.axis_index("s") * NC + lax.axis_index("c")
        pltpu.sync_copy(nid_hbm.at[pl.ds(wid * rw, rw)], nidx_v)
        pltpu.sync_copy(oid_hbm.at[pl.ds(wid * rw, rw)], oidx_v)
        pltpu.sync_copy(xid_hbm.at[pl.ds(wid * rw * NS, rw * NS)], xidx_v)

        lane = lax.iota(jnp.int32, GRP)

        def fire(c, slot):
            pltpu.async_copy(
                node_hbm.at[nidx_v.at[pl.ds(c * CH, CH)]],
                node_v.at[slot], sem)
            pltpu.async_copy(
                ctx_hbm.at[oidx_v.at[pl.ds(c * CH, CH)]],
                out_v.at[slot], sem)
            pltpu.async_copy(
                ctx_hbm.at[xidx_v.at[pl.ds(c * CH * NS, CH * NS)]],
                noise_v.at[slot], sem)

        def wait_chunk(c, slot):
            pltpu.make_async_copy(
                node_hbm.at[nidx_v.at[pl.ds(c * CH, CH)]],
                node_v.at[slot], sem).wait()
            pltpu.make_async_copy(
                ctx_hbm.at[oidx_v.at[pl.ds(c * CH, CH)]],
                out_v.at[slot], sem).wait()
            pltpu.make_async_copy(
                ctx_hbm.at[xidx_v.at[pl.ds(c * CH * NS, CH * NS)]],
                noise_v.at[slot], sem).wait()

        fire(0, 0)

        def chunk_body(c, carry):
            slot = lax.rem(c, 2)

            @pl.when(c + 1 < nchunk)
            def _():
                fire(c + 1, 1 - slot)

            wait_chunk(c, slot)
            svec = jnp.full((GRP,), 0, jnp.int32) + slot

            def group_body(g, gcarry):
                row16 = g * GRP + lane
                nrows = [row16 * NS + s for s in range(NS)]
                accs = [jnp.zeros((GRP,), jnp.float32) for _ in range(6)]
                for d in range(D):
                    dcol = jnp.full((GRP,), d, jnp.int32)
                    vi = plsc.load_gather(node_v, [svec, row16, dcol])
                    vo = plsc.load_gather(out_v, [svec, row16, dcol])
                    accs[0] = accs[0] + vi * vo
                    for s in range(NS):
                        vn = plsc.load_gather(
                            noise_v, [svec, nrows[s], dcol])
                        accs[1 + s] = accs[1 + s] + vi * vn
                base = c * CH + g * GRP
                for k in range(6):
                    t_v[pl.ds(k * rw + base, GRP)] = accs[k]
                return gcarry

            lax.fori_loop(0, CH // GRP, group_body, 0)
            return carry

        lax.fori_loop(0, nchunk, chunk_body, 0)
        for k in range(6):
            pltpu.sync_copy(t_v.at[pl.ds(k * rw, rw)],
                            t_hbm.at[k, pl.ds(wid * rw, rw)])

    return body(node_table, ctx_table, nid, oid, xid)


def _tc_reduce(t, batch):
    def body(t_ref, o_ref):
        x = t_ref[...]
        pos = x[0:1, :]
        neg = x[1:6, :]

        def logsig(z):
            return jnp.minimum(z, 0.0) - jnp.log(1.0 + jnp.exp(-jnp.abs(z)))

        total = jnp.sum(logsig(pos)) + jnp.sum(logsig(-neg))
        o_ref[0, 0] = -total / batch

    out = pl.pallas_call(
        body,
        out_shape=jax.ShapeDtypeStruct((1, 1), jnp.float32),
        out_specs=pl.BlockSpec(memory_space=pltpu.SMEM),
    )(t)
    return out[0, 0]


def kernel(input_labels, out_labels, noise_idx, num_sampled, node_table,
           ctx_table):
    b, w = out_labels.shape
    r_total = b * w
    nid = jnp.tile(input_labels.astype(jnp.int32), w)
    oid = out_labels.reshape(-1).astype(jnp.int32)
    xid = noise_idx.astype(jnp.int32).reshape(-1)
    t = _sc_logits(node_table, ctx_table, nid, oid, xid, r_total)
    return _tc_reduce(t, b)
